# Initial kernel scaffold; baseline (speedup 1.0000x reference)
import jax
import jax.numpy as jnp
from jax.experimental import pallas as pl


def kernel(x_question, x_answer, edge_index_qa, edge_index_aq, W_src_qa, W_dst_qa, att_src_qa, att_dst_qa, bias_qa, W_src_aq, W_dst_aq, att_src_aq, att_dst_aq, bias_aq, W_out, b_out, ew_qa, ew_aq):
    raise NotImplementedError("write your pallas kernel here")


# speedup vs baseline: 23.1076x; 23.1076x over previous
import functools
import jax
import jax.numpy as jnp
from jax import lax
from jax.experimental import pallas as pl
from jax.experimental.pallas import tpu as pltpu, tpu_sc as plsc

N = 10000
E = 160000
D_IN = 128
HID = 64
HEADS = 8
NC_OUT = 4

NP = 10240
EP = 163840
PAD_NODE = N

NUM_TILES = 32
EDGES_PER_TILE = EP // NUM_TILES
CHUNK = 128
NCHUNKS = EDGES_PER_TILE // CHUNK
ROWS_PER_TILE = NP // 16

BN = 256
GRID_N = NP // BN


def _tc1_body(xq, xa, wsqa, asqa, wdqa, adqa, wsaq, asaq, wdaq, adaq,
              hs_qa, al_s_qa, al_d_qa, hs_aq, al_s_aq, al_d_aq):
    xqb = xq[...]
    xab = xa[...]
    hq = jnp.dot(xqb, wsqa[...], preferred_element_type=jnp.float32)
    hs_qa[...] = hq
    al_s_qa[...] = jnp.dot(hq, asqa[...], preferred_element_type=jnp.float32)
    vd_qa = jnp.dot(wdqa[...], adqa[...], preferred_element_type=jnp.float32)
    al_d_qa[...] = jnp.dot(xab, vd_qa, preferred_element_type=jnp.float32)
    ha = jnp.dot(xab, wsaq[...], preferred_element_type=jnp.float32)
    hs_aq[...] = ha
    al_s_aq[...] = jnp.dot(ha, asaq[...], preferred_element_type=jnp.float32)
    vd_aq = jnp.dot(wdaq[...], adaq[...], preferred_element_type=jnp.float32)
    al_d_aq[...] = jnp.dot(xqb, vd_aq, preferred_element_type=jnp.float32)


def _sc_phase_a_body(es_qa, ed_qa, es_aq, ed_aq, asq, adq, asa, ada, z16,
                     ee_qa, ee_aq, dp_qa, dp_aq,
                     idx_s, idx_d, abuf, bbuf, ebuf, den_sh, sem1, sem2):
    cid = lax.axis_index("c")
    sid = lax.axis_index("s")
    wid = cid * 16 + sid
    r0 = sid * ROWS_PER_TILE

    for es, ed, asrc, adst, ee_o, dp_o in (
        (es_qa, ed_qa, asq, adq, ee_qa, dp_qa),
        (es_aq, ed_aq, asa, ada, ee_aq, dp_aq),
    ):
        pltpu.sync_copy(z16.at[pl.ds(r0, ROWS_PER_TILE)],
                        den_sh.at[pl.ds(r0, ROWS_PER_TILE)])
        plsc.subcore_barrier()

        def chunk(k, _, es=es, ed=ed, asrc=asrc, adst=adst, ee_o=ee_o):
            base = wid * EDGES_PER_TILE + k * CHUNK
            pltpu.sync_copy(es.at[pl.ds(base, CHUNK)], idx_s)
            pltpu.sync_copy(ed.at[pl.ds(base, CHUNK)], idx_d)
            cp1 = pltpu.async_copy(asrc.at[idx_s], abuf, sem1)
            cp2 = pltpu.async_copy(adst.at[idx_d], bbuf, sem2)
            cp1.wait()
            cp2.wait()

            def row(i, _):
                s = abuf[i, :] + bbuf[i, :]
                e = jnp.maximum(s, 0.2 * s)
                ebuf[i, :] = jnp.exp(e)
                return 0
            lax.fori_loop(0, CHUNK, row, 0)
            pltpu.sync_copy(ebuf, ee_o.at[pl.ds(base, CHUNK)])
            pltpu.sync_copy(ebuf, den_sh.at[idx_d], add=True)
            return 0

        lax.fori_loop(0, NCHUNKS, chunk, 0)
        plsc.subcore_barrier()
        pltpu.sync_copy(den_sh.at[pl.ds(r0, ROWS_PER_TILE)],
                        dp_o.at[cid, pl.ds(r0, ROWS_PER_TILE)])
        plsc.subcore_barrier()


def _sc_phase_b_body(es_qa, ed_qa, es_aq, ed_aq, ee_qa, ee_aq, den_qa, den_aq,
                     hs_qa, hs_aq, z64,
                     op_qa, op_aq,
                     idx_s, idx_d, eebuf, denbuf, wbuf, hsbuf, msgbuf,
                     out_sh, sem1, sem2):
    cid = lax.axis_index("c")
    sid = lax.axis_index("s")
    wid = cid * 16 + sid
    r0 = sid * ROWS_PER_TILE

    for es, ed, ee, den, hs, op_o in (
        (es_qa, ed_qa, ee_qa, den_qa, hs_qa, op_qa),
        (es_aq, ed_aq, ee_aq, den_aq, hs_aq, op_aq),
    ):
        pltpu.sync_copy(z64.at[pl.ds(r0, ROWS_PER_TILE)],
                        out_sh.at[pl.ds(r0, ROWS_PER_TILE)])
        plsc.subcore_barrier()

        def chunk(k, _, es=es, ed=ed, ee=ee, den=den, hs=hs):
            base = wid * EDGES_PER_TILE + k * CHUNK
            pltpu.sync_copy(es.at[pl.ds(base, CHUNK)], idx_s)
            pltpu.sync_copy(ed.at[pl.ds(base, CHUNK)], idx_d)
            cp1 = pltpu.async_copy(hs.at[idx_s], hsbuf, sem1)
            cp2 = pltpu.async_copy(den.at[idx_d], denbuf, sem2)
            pltpu.sync_copy(ee.at[pl.ds(base, CHUNK)], eebuf)
            cp2.wait()

            def wrow(i, _):
                wbuf[i, :] = eebuf[i, :] * 0.125 / (denbuf[i, :] + 1e-16)
                return 0
            lax.fori_loop(0, CHUNK, wrow, 0)
            cp1.wait()

            def edge(e_, _):
                wrow = wbuf[e_, :]
                for j in range(4):
                    acc = wrow[0] * hsbuf[e_, pl.ds(j * 16, 16)]
                    for h in range(1, 8):
                        acc = acc + wrow[h] * hsbuf[e_, pl.ds(h * 64 + j * 16, 16)]
                    msgbuf[e_, pl.ds(j * 16, 16)] = acc
                return 0
            lax.fori_loop(0, CHUNK, edge, 0)
            pltpu.sync_copy(msgbuf, out_sh.at[idx_d], add=True)
            return 0

        lax.fori_loop(0, NCHUNKS, chunk, 0)
        plsc.subcore_barrier()
        pltpu.sync_copy(out_sh.at[pl.ds(r0, ROWS_PER_TILE)],
                        op_o.at[cid, pl.ds(r0, ROWS_PER_TILE)])
        plsc.subcore_barrier()


def _tc2_body(opqa, opaq, bqa, baq, wo, bo, out):
    f1 = opaq[0] + opaq[1] + baq[...]
    f2 = opqa[0] + opqa[1] + bqa[...]
    out[...] = (jnp.dot(f1, wo[0:64, :], preferred_element_type=jnp.float32)
                + jnp.dot(f2, wo[64:128, :], preferred_element_type=jnp.float32)
                + bo[...])


def _full(shape):
    return pl.BlockSpec(shape, lambda i: (0,) * len(shape))


@jax.jit
def kernel(x_question, x_answer, edge_index_qa, edge_index_aq,
           W_src_qa, W_dst_qa, att_src_qa, att_dst_qa, bias_qa,
           W_src_aq, W_dst_aq, att_src_aq, att_dst_aq, bias_aq,
           W_out, b_out, ew_qa, ew_aq):
    f32 = jnp.float32
    eye = jnp.eye(HEADS, dtype=f32)

    def amat(att):
        a = (att[:, :, None] * eye[:, None, :]).reshape(HEADS * HID, HEADS)
        return jnp.pad(a, ((0, 0), (0, 16 - HEADS)))

    a_s_qa, a_d_qa = amat(att_src_qa), amat(att_dst_qa)
    a_s_aq, a_d_aq = amat(att_src_aq), amat(att_dst_aq)

    xq = jnp.pad(x_question, ((0, NP - N), (0, 0)))
    xa = jnp.pad(x_answer, ((0, NP - N), (0, 0)))

    pad_idx = jnp.full((EP - E,), PAD_NODE, dtype=jnp.int32)
    es_qa = jnp.concatenate([edge_index_qa[0].astype(jnp.int32), pad_idx])
    ed_qa = jnp.concatenate([edge_index_qa[1].astype(jnp.int32), pad_idx])
    es_aq = jnp.concatenate([edge_index_aq[0].astype(jnp.int32), pad_idx])
    ed_aq = jnp.concatenate([edge_index_aq[1].astype(jnp.int32), pad_idx])

    tc1 = pl.pallas_call(
        _tc1_body,
        grid=(GRID_N,),
        in_specs=[
            pl.BlockSpec((BN, D_IN), lambda i: (i, 0)),
            pl.BlockSpec((BN, D_IN), lambda i: (i, 0)),
            _full((D_IN, HEADS * HID)), _full((HEADS * HID, 16)),
            _full((D_IN, HEADS * HID)), _full((HEADS * HID, 16)),
            _full((D_IN, HEADS * HID)), _full((HEADS * HID, 16)),
            _full((D_IN, HEADS * HID)), _full((HEADS * HID, 16)),
        ],
        out_specs=[
            pl.BlockSpec((BN, HEADS * HID), lambda i: (i, 0)),
            pl.BlockSpec((BN, 16), lambda i: (i, 0)),
            pl.BlockSpec((BN, 16), lambda i: (i, 0)),
            pl.BlockSpec((BN, HEADS * HID), lambda i: (i, 0)),
            pl.BlockSpec((BN, 16), lambda i: (i, 0)),
            pl.BlockSpec((BN, 16), lambda i: (i, 0)),
        ],
        out_shape=[
            jax.ShapeDtypeStruct((NP, HEADS * HID), f32),
            jax.ShapeDtypeStruct((NP, 16), f32),
            jax.ShapeDtypeStruct((NP, 16), f32),
            jax.ShapeDtypeStruct((NP, HEADS * HID), f32),
            jax.ShapeDtypeStruct((NP, 16), f32),
            jax.ShapeDtypeStruct((NP, 16), f32),
        ],
    )
    hs_qa, al_s_qa, al_d_qa, hs_aq, al_s_aq, al_d_aq = tc1(
        xq, xa, W_src_qa, a_s_qa, W_dst_qa, a_d_qa,
        W_src_aq, a_s_aq, W_dst_aq, a_d_aq)

    mesh = plsc.VectorSubcoreMesh(core_axis_name="c", subcore_axis_name="s")
    z16 = jnp.zeros((NP, 16), f32)
    z64 = jnp.zeros((NP, HID), f32)

    phase_a = pl.kernel(
        _sc_phase_a_body,
        out_type=[
            jax.ShapeDtypeStruct((EP, 16), f32),
            jax.ShapeDtypeStruct((EP, 16), f32),
            jax.ShapeDtypeStruct((2, NP, 16), f32),
            jax.ShapeDtypeStruct((2, NP, 16), f32),
        ],
        mesh=mesh,
        compiler_params=pltpu.CompilerParams(use_tc_tiling_on_sc=False),
        scratch_types=[
            pltpu.VMEM((CHUNK,), jnp.int32),
            pltpu.VMEM((CHUNK,), jnp.int32),
            pltpu.VMEM((CHUNK, 16), f32),
            pltpu.VMEM((CHUNK, 16), f32),
            pltpu.VMEM((CHUNK, 16), f32),
            pltpu.VMEM_SHARED((NP, 16), f32),
            pltpu.SemaphoreType.DMA,
            pltpu.SemaphoreType.DMA,
        ],
    )
    ee_qa, ee_aq, dp_qa, dp_aq = phase_a(
        es_qa, ed_qa, es_aq, ed_aq, al_s_qa, al_d_qa, al_s_aq, al_d_aq, z16)

    den_qa = dp_qa[0] + dp_qa[1]
    den_aq = dp_aq[0] + dp_aq[1]

    phase_b = pl.kernel(
        _sc_phase_b_body,
        out_type=[
            jax.ShapeDtypeStruct((2, NP, HID), f32),
            jax.ShapeDtypeStruct((2, NP, HID), f32),
        ],
        mesh=mesh,
        compiler_params=pltpu.CompilerParams(use_tc_tiling_on_sc=False),
        scratch_types=[
            pltpu.VMEM((CHUNK,), jnp.int32),
            pltpu.VMEM((CHUNK,), jnp.int32),
            pltpu.VMEM((CHUNK, 16), f32),
            pltpu.VMEM((CHUNK, 16), f32),
            pltpu.VMEM((CHUNK, 16), f32),
            pltpu.VMEM((CHUNK, HEADS * HID), f32),
            pltpu.VMEM((CHUNK, HID), f32),
            pltpu.VMEM_SHARED((NP, HID), f32),
            pltpu.SemaphoreType.DMA,
            pltpu.SemaphoreType.DMA,
        ],
    )
    op_qa, op_aq = phase_b(
        es_qa, ed_qa, es_aq, ed_aq, ee_qa, ee_aq, den_qa, den_aq,
        hs_qa, hs_aq, z64)

    tc2 = pl.pallas_call(
        _tc2_body,
        grid=(GRID_N,),
        in_specs=[
            pl.BlockSpec((2, BN, HID), lambda i: (0, i, 0)),
            pl.BlockSpec((2, BN, HID), lambda i: (0, i, 0)),
            _full((1, HID)), _full((1, HID)),
            _full((2 * HID, NC_OUT)), _full((1, NC_OUT)),
        ],
        out_specs=pl.BlockSpec((BN, NC_OUT), lambda i: (i, 0)),
        out_shape=jax.ShapeDtypeStruct((NP, NC_OUT), f32),
    )
    preds = tc2(op_qa, op_aq, bias_qa.reshape(1, HID), bias_aq.reshape(1, HID),
                W_out, b_out.reshape(1, NC_OUT))
    return (preds[:N], ew_qa, ew_aq)


# speedup vs baseline: 33.5705x; 1.4528x over previous
import functools
import jax
import jax.numpy as jnp
from jax import lax
from jax.experimental import pallas as pl
from jax.experimental.pallas import tpu as pltpu, tpu_sc as plsc

N = 10000
E = 160000
D_IN = 128
HID = 64
HEADS = 8
NC_OUT = 4

NP = 10240
EP = 163840
PAD_NODE = N

NUM_TILES = 32
EDGES_PER_TILE = EP // NUM_TILES
CHUNK = 128
NCHUNKS = EDGES_PER_TILE // CHUNK
CHUNK_B = 64
NCHUNKS_B = EDGES_PER_TILE // CHUNK_B
ROWS_PER_TILE = NP // 16

BN = 256
GRID_N = NP // BN


def _tc1_body(xq, xa, wsqa, asqa, wdqa, adqa, wsaq, asaq, wdaq, adaq,
              hs_qa, al_s_qa, al_d_qa, hs_aq, al_s_aq, al_d_aq):
    xqb = xq[...]
    xab = xa[...]
    hq = jnp.dot(xqb, wsqa[...], preferred_element_type=jnp.float32)
    hs_qa[...] = hq
    al_s_qa[...] = jnp.dot(hq, asqa[...], preferred_element_type=jnp.float32)
    vd_qa = jnp.dot(wdqa[...], adqa[...], preferred_element_type=jnp.float32)
    al_d_qa[...] = jnp.dot(xab, vd_qa, preferred_element_type=jnp.float32)
    ha = jnp.dot(xab, wsaq[...], preferred_element_type=jnp.float32)
    hs_aq[...] = ha
    al_s_aq[...] = jnp.dot(ha, asaq[...], preferred_element_type=jnp.float32)
    vd_aq = jnp.dot(wdaq[...], adaq[...], preferred_element_type=jnp.float32)
    al_d_aq[...] = jnp.dot(xqb, vd_aq, preferred_element_type=jnp.float32)


def _sc_phase_a_body(es_qa, ed_qa, es_aq, ed_aq, asq, adq, asa, ada, z16,
                     ee_qa, ee_aq, dp_qa, dp_aq,
                     idx_s, idx_d, abuf, bbuf, ebuf, den_sh, sem1, sem2):
    cid = lax.axis_index("c")
    sid = lax.axis_index("s")
    wid = cid * 16 + sid
    r0 = sid * ROWS_PER_TILE

    for es, ed, asrc, adst, ee_o, dp_o in (
        (es_qa, ed_qa, asq, adq, ee_qa, dp_qa),
        (es_aq, ed_aq, asa, ada, ee_aq, dp_aq),
    ):
        pltpu.sync_copy(z16.at[pl.ds(r0, ROWS_PER_TILE)],
                        den_sh.at[pl.ds(r0, ROWS_PER_TILE)])
        plsc.subcore_barrier()

        def chunk(k, _, es=es, ed=ed, asrc=asrc, adst=adst, ee_o=ee_o):
            base = wid * EDGES_PER_TILE + k * CHUNK
            pltpu.sync_copy(es.at[pl.ds(base, CHUNK)], idx_s)
            pltpu.sync_copy(ed.at[pl.ds(base, CHUNK)], idx_d)
            cp1 = pltpu.async_copy(asrc.at[idx_s], abuf, sem1)
            cp2 = pltpu.async_copy(adst.at[idx_d], bbuf, sem2)
            cp1.wait()
            cp2.wait()

            def row(i, _):
                s = abuf[i, :] + bbuf[i, :]
                e = jnp.maximum(s, 0.2 * s)
                ebuf[i, :] = jnp.exp(e)
                return 0
            lax.fori_loop(0, CHUNK, row, 0)
            pltpu.sync_copy(ebuf, ee_o.at[pl.ds(base, CHUNK)])
            pltpu.sync_copy(ebuf, den_sh.at[idx_d], add=True)
            return 0

        lax.fori_loop(0, NCHUNKS, chunk, 0)
        plsc.subcore_barrier()
        pltpu.sync_copy(den_sh.at[pl.ds(r0, ROWS_PER_TILE)],
                        dp_o.at[cid, pl.ds(r0, ROWS_PER_TILE)])
        plsc.subcore_barrier()


def _sc_phase_b_body(ep_qa, ep_aq, ee_qa, ee_aq, den_qa, den_aq,
                     hs_qa, hs_aq, z64,
                     op_qa, op_aq,
                     idx2_0, idx2_1, ee_0, ee_1, den_0, den_1, wbuf,
                     hs_0, hs_1, msg_0, msg_1,
                     out_sh,
                     sem_i0, sem_i1, sem_e0, sem_e1, sem_d0, sem_d1,
                     sem_h0, sem_h1, sem_s0, sem_s1):
    cid = lax.axis_index("c")
    sid = lax.axis_index("s")
    wid = cid * 16 + sid
    r0 = sid * ROWS_PER_TILE
    idx2 = (idx2_0, idx2_1)
    eeb = (ee_0, ee_1)
    denb = (den_0, den_1)
    hsb = (hs_0, hs_1)
    msgb = (msg_0, msg_1)
    sem_i = (sem_i0, sem_i1)
    sem_e = (sem_e0, sem_e1)
    sem_d = (sem_d0, sem_d1)
    sem_h = (sem_h0, sem_h1)
    sem_s = (sem_s0, sem_s1)

    for ep, ee, den, hs, op_o in (
        (ep_qa, ee_qa, den_qa, hs_qa, op_qa),
        (ep_aq, ee_aq, den_aq, hs_aq, op_aq),
    ):
        pltpu.sync_copy(z64.at[pl.ds(r0, ROWS_PER_TILE)],
                        out_sh.at[pl.ds(r0, ROWS_PER_TILE)])
        plsc.subcore_barrier()

        def fire(k, b, ep=ep, ee=ee, den=den, hs=hs):
            g = wid * NCHUNKS_B + k
            base = g * CHUNK_B
            pltpu.sync_copy(ep.at[g], idx2[b])
            pltpu.make_async_copy(hs.at[idx2[b].at[0]], hsb[b], sem_h[b]).start()
            pltpu.make_async_copy(den.at[idx2[b].at[1]], denb[b], sem_d[b]).start()
            pltpu.make_async_copy(ee.at[pl.ds(base, CHUNK_B)], eeb[b],
                                  sem_e[b]).start()

        def wait_scatter(b):
            pltpu.make_async_copy(msgb[b], out_sh.at[idx2[b].at[1]],
                                  sem_s[b]).wait()

        def process(k, b, ee=ee, den=den, hs=hs):
            pltpu.make_async_copy(den.at[idx2[b].at[1]], denb[b], sem_d[b]).wait()
            pltpu.make_async_copy(ee.at[pl.ds(0, CHUNK_B)], eeb[b], sem_e[b]).wait()

            def wrow(i, _):
                wbuf[i, :] = eeb[b][i, :] * 0.125 / (denb[b][i, :] + 1e-16)
                return 0
            lax.fori_loop(0, CHUNK_B, wrow, 0)

            @pl.when(k > 0)
            def _():
                wait_scatter(1 - b)

            @pl.when(k + 1 < NCHUNKS_B)
            def _():
                fire(k + 1, 1 - b)

            pltpu.make_async_copy(hs.at[idx2[b].at[0]], hsb[b], sem_h[b]).wait()

            def edge(e_, _):
                wrow_ = wbuf[e_, :]
                for j in range(4):
                    acc = wrow_[0] * hsb[b][e_, pl.ds(j * 16, 16)]
                    for h in range(1, 8):
                        acc = acc + wrow_[h] * hsb[b][e_, pl.ds(h * 64 + j * 16, 16)]
                    msgb[b][e_, pl.ds(j * 16, 16)] = acc
                return 0
            lax.fori_loop(0, CHUNK_B, edge, 0)
            pltpu.make_async_copy(msgb[b], out_sh.at[idx2[b].at[1]],
                                  sem_s[b]).start(add=True)

        fire(0, 0)

        def pair(kk, _):
            process(2 * kk, 0)
            process(2 * kk + 1, 1)
            return 0
        lax.fori_loop(0, NCHUNKS_B // 2, pair, 0)
        wait_scatter(1)

        plsc.subcore_barrier()
        pltpu.sync_copy(out_sh.at[pl.ds(r0, ROWS_PER_TILE)],
                        op_o.at[cid, pl.ds(r0, ROWS_PER_TILE)])
        plsc.subcore_barrier()


def _tc2_body(opqa, opaq, bqa, baq, wo, bo, out):
    f1 = opaq[0] + opaq[1] + baq[...]
    f2 = opqa[0] + opqa[1] + bqa[...]
    out[...] = (jnp.dot(f1, wo[0:64, :], preferred_element_type=jnp.float32)
                + jnp.dot(f2, wo[64:128, :], preferred_element_type=jnp.float32)
                + bo[...])


def _full(shape):
    return pl.BlockSpec(shape, lambda i: (0,) * len(shape))


@jax.jit
def kernel(x_question, x_answer, edge_index_qa, edge_index_aq,
           W_src_qa, W_dst_qa, att_src_qa, att_dst_qa, bias_qa,
           W_src_aq, W_dst_aq, att_src_aq, att_dst_aq, bias_aq,
           W_out, b_out, ew_qa, ew_aq):
    f32 = jnp.float32
    eye = jnp.eye(HEADS, dtype=f32)

    def amat(att):
        a = (att[:, :, None] * eye[:, None, :]).reshape(HEADS * HID, HEADS)
        return jnp.pad(a, ((0, 0), (0, 16 - HEADS)))

    a_s_qa, a_d_qa = amat(att_src_qa), amat(att_dst_qa)
    a_s_aq, a_d_aq = amat(att_src_aq), amat(att_dst_aq)

    xq = jnp.pad(x_question, ((0, NP - N), (0, 0)))
    xa = jnp.pad(x_answer, ((0, NP - N), (0, 0)))

    pad_idx = jnp.full((EP - E,), PAD_NODE, dtype=jnp.int32)
    es_qa = jnp.concatenate([edge_index_qa[0].astype(jnp.int32), pad_idx])
    ed_qa = jnp.concatenate([edge_index_qa[1].astype(jnp.int32), pad_idx])
    es_aq = jnp.concatenate([edge_index_aq[0].astype(jnp.int32), pad_idx])
    ed_aq = jnp.concatenate([edge_index_aq[1].astype(jnp.int32), pad_idx])
    ep_qa = jnp.stack([es_qa.reshape(-1, CHUNK_B), ed_qa.reshape(-1, CHUNK_B)], 1)
    ep_aq = jnp.stack([es_aq.reshape(-1, CHUNK_B), ed_aq.reshape(-1, CHUNK_B)], 1)

    tc1 = pl.pallas_call(
        _tc1_body,
        grid=(GRID_N,),
        in_specs=[
            pl.BlockSpec((BN, D_IN), lambda i: (i, 0)),
            pl.BlockSpec((BN, D_IN), lambda i: (i, 0)),
            _full((D_IN, HEADS * HID)), _full((HEADS * HID, 16)),
            _full((D_IN, HEADS * HID)), _full((HEADS * HID, 16)),
            _full((D_IN, HEADS * HID)), _full((HEADS * HID, 16)),
            _full((D_IN, HEADS * HID)), _full((HEADS * HID, 16)),
        ],
        out_specs=[
            pl.BlockSpec((BN, HEADS * HID), lambda i: (i, 0)),
            pl.BlockSpec((BN, 16), lambda i: (i, 0)),
            pl.BlockSpec((BN, 16), lambda i: (i, 0)),
            pl.BlockSpec((BN, HEADS * HID), lambda i: (i, 0)),
            pl.BlockSpec((BN, 16), lambda i: (i, 0)),
            pl.BlockSpec((BN, 16), lambda i: (i, 0)),
        ],
        out_shape=[
            jax.ShapeDtypeStruct((NP, HEADS * HID), f32),
            jax.ShapeDtypeStruct((NP, 16), f32),
            jax.ShapeDtypeStruct((NP, 16), f32),
            jax.ShapeDtypeStruct((NP, HEADS * HID), f32),
            jax.ShapeDtypeStruct((NP, 16), f32),
            jax.ShapeDtypeStruct((NP, 16), f32),
        ],
    )
    hs_qa, al_s_qa, al_d_qa, hs_aq, al_s_aq, al_d_aq = tc1(
        xq, xa, W_src_qa, a_s_qa, W_dst_qa, a_d_qa,
        W_src_aq, a_s_aq, W_dst_aq, a_d_aq)

    mesh = plsc.VectorSubcoreMesh(core_axis_name="c", subcore_axis_name="s")
    z16 = jnp.zeros((NP, 16), f32)
    z64 = jnp.zeros((NP, HID), f32)

    phase_a = pl.kernel(
        _sc_phase_a_body,
        out_type=[
            jax.ShapeDtypeStruct((EP, 16), f32),
            jax.ShapeDtypeStruct((EP, 16), f32),
            jax.ShapeDtypeStruct((2, NP, 16), f32),
            jax.ShapeDtypeStruct((2, NP, 16), f32),
        ],
        mesh=mesh,
        compiler_params=pltpu.CompilerParams(use_tc_tiling_on_sc=False),
        scratch_types=[
            pltpu.VMEM((CHUNK,), jnp.int32),
            pltpu.VMEM((CHUNK,), jnp.int32),
            pltpu.VMEM((CHUNK, 16), f32),
            pltpu.VMEM((CHUNK, 16), f32),
            pltpu.VMEM((CHUNK, 16), f32),
            pltpu.VMEM_SHARED((NP, 16), f32),
            pltpu.SemaphoreType.DMA,
            pltpu.SemaphoreType.DMA,
        ],
    )
    ee_qa, ee_aq, dp_qa, dp_aq = phase_a(
        es_qa, ed_qa, es_aq, ed_aq, al_s_qa, al_d_qa, al_s_aq, al_d_aq, z16)

    den_qa = dp_qa[0] + dp_qa[1]
    den_aq = dp_aq[0] + dp_aq[1]

    phase_b = pl.kernel(
        _sc_phase_b_body,
        out_type=[
            jax.ShapeDtypeStruct((2, NP, HID), f32),
            jax.ShapeDtypeStruct((2, NP, HID), f32),
        ],
        mesh=mesh,
        compiler_params=pltpu.CompilerParams(use_tc_tiling_on_sc=False),
        scratch_types=[
            pltpu.VMEM((2, CHUNK_B), jnp.int32),
            pltpu.VMEM((2, CHUNK_B), jnp.int32),
            pltpu.VMEM((CHUNK_B, 16), f32),
            pltpu.VMEM((CHUNK_B, 16), f32),
            pltpu.VMEM((CHUNK_B, 16), f32),
            pltpu.VMEM((CHUNK_B, 16), f32),
            pltpu.VMEM((CHUNK_B, 16), f32),
            pltpu.VMEM((CHUNK_B, HEADS * HID), f32),
            pltpu.VMEM((CHUNK_B, HEADS * HID), f32),
            pltpu.VMEM((CHUNK_B, HID), f32),
            pltpu.VMEM((CHUNK_B, HID), f32),
            pltpu.VMEM_SHARED((NP, HID), f32),
        ] + [pltpu.SemaphoreType.DMA] * 10,
    )
    op_qa, op_aq = phase_b(
        ep_qa, ep_aq, ee_qa, ee_aq, den_qa, den_aq,
        hs_qa, hs_aq, z64)

    tc2 = pl.pallas_call(
        _tc2_body,
        grid=(GRID_N,),
        in_specs=[
            pl.BlockSpec((2, BN, HID), lambda i: (0, i, 0)),
            pl.BlockSpec((2, BN, HID), lambda i: (0, i, 0)),
            _full((1, HID)), _full((1, HID)),
            _full((2 * HID, NC_OUT)), _full((1, NC_OUT)),
        ],
        out_specs=pl.BlockSpec((BN, NC_OUT), lambda i: (i, 0)),
        out_shape=jax.ShapeDtypeStruct((NP, NC_OUT), f32),
    )
    preds = tc2(op_qa, op_aq, bias_qa.reshape(1, HID), bias_aq.reshape(1, HID),
                W_out, b_out.reshape(1, NC_OUT))
    return (preds[:N], ew_qa, ew_aq)


# speedup vs baseline: 40.4837x; 1.2059x over previous
import functools
import jax
import jax.numpy as jnp
from jax import lax
from jax.experimental import pallas as pl
from jax.experimental.pallas import tpu as pltpu, tpu_sc as plsc

N = 10000
E = 160000
D_IN = 128
HID = 64
HEADS = 8
NC_OUT = 4

NP = 10240
EP = 163840
PAD_NODE = N

NUM_TILES = 32
EDGES_PER_TILE = EP // NUM_TILES
CHUNK = 128
NCHUNKS = EDGES_PER_TILE // CHUNK
CHUNK_B = 64
NCHUNKS_B = EDGES_PER_TILE // CHUNK_B
ROWS_PER_TILE = NP // 16

BN = 256
GRID_N = NP // BN


def _tc1_body(xq, xa, wsqa, asqa, wdqa, adqa, wsaq, asaq, wdaq, adaq,
              hs_qa, al_s_qa, al_d_qa, hs_aq, al_s_aq, al_d_aq):
    xqb = xq[...]
    xab = xa[...]
    hq = jnp.dot(xqb, wsqa[...], preferred_element_type=jnp.float32)
    hs_qa[...] = hq
    al_s_qa[...] = jnp.dot(hq, asqa[...], preferred_element_type=jnp.float32)
    vd_qa = jnp.dot(wdqa[...], adqa[...], preferred_element_type=jnp.float32)
    al_d_qa[...] = jnp.dot(xab, vd_qa, preferred_element_type=jnp.float32)
    ha = jnp.dot(xab, wsaq[...], preferred_element_type=jnp.float32)
    hs_aq[...] = ha
    al_s_aq[...] = jnp.dot(ha, asaq[...], preferred_element_type=jnp.float32)
    vd_aq = jnp.dot(wdaq[...], adaq[...], preferred_element_type=jnp.float32)
    al_d_aq[...] = jnp.dot(xqb, vd_aq, preferred_element_type=jnp.float32)


def _sc_phase_a_body(epa_qa, epa_aq, asq, adq, asa, ada, z16,
                     ee_qa, ee_aq, dp_qa, dp_aq,
                     idx2_0, idx2_1, a_0, a_1, b_0, b_1, e_0, e_1, den_sh,
                     sem_i0, sem_i1, sem_a0, sem_a1, sem_b0, sem_b1,
                     sem_w0, sem_w1, sem_s0, sem_s1):
    cid = lax.axis_index("c")
    sid = lax.axis_index("s")
    wid = cid * 16 + sid
    r0 = sid * ROWS_PER_TILE
    idx2 = (idx2_0, idx2_1)
    ab = (a_0, a_1)
    bb = (b_0, b_1)
    eb = (e_0, e_1)
    sem_a = (sem_a0, sem_a1)
    sem_b = (sem_b0, sem_b1)
    sem_w = (sem_w0, sem_w1)
    sem_s = (sem_s0, sem_s1)

    for epa, asrc, adst, ee_o, dp_o in (
        (epa_qa, asq, adq, ee_qa, dp_qa),
        (epa_aq, asa, ada, ee_aq, dp_aq),
    ):
        pltpu.sync_copy(z16.at[pl.ds(r0, ROWS_PER_TILE)],
                        den_sh.at[pl.ds(r0, ROWS_PER_TILE)])
        plsc.subcore_barrier()

        def fire(k, b, epa=epa, asrc=asrc, adst=adst):
            g = wid * NCHUNKS + k
            pltpu.sync_copy(epa.at[g], idx2[b])
            pltpu.make_async_copy(asrc.at[idx2[b].at[0]], ab[b], sem_a[b]).start()
            pltpu.make_async_copy(adst.at[idx2[b].at[1]], bb[b], sem_b[b]).start()

        def drain(k, b, ee_o=ee_o):
            base = (wid * NCHUNKS + k) * CHUNK
            pltpu.make_async_copy(eb[b], ee_o.at[pl.ds(base, CHUNK)],
                                  sem_w[b]).wait()
            pltpu.make_async_copy(eb[b], den_sh.at[idx2[b].at[1]],
                                  sem_s[b]).wait()

        def process(k, b, asrc=asrc, adst=adst, ee_o=ee_o):
            base = (wid * NCHUNKS + k) * CHUNK
            pltpu.make_async_copy(asrc.at[idx2[b].at[0]], ab[b], sem_a[b]).wait()
            pltpu.make_async_copy(adst.at[idx2[b].at[1]], bb[b], sem_b[b]).wait()

            @pl.when(k > 0)
            def _():
                drain(k - 1, 1 - b)

            @pl.when(k + 1 < NCHUNKS)
            def _():
                fire(k + 1, 1 - b)

            def row(i, _):
                s = ab[b][i, :] + bb[b][i, :]
                e = jnp.maximum(s, 0.2 * s)
                eb[b][i, :] = jnp.exp(e)
                return 0
            lax.fori_loop(0, CHUNK, row, 0)
            pltpu.make_async_copy(eb[b], ee_o.at[pl.ds(base, CHUNK)],
                                  sem_w[b]).start()
            pltpu.make_async_copy(eb[b], den_sh.at[idx2[b].at[1]],
                                  sem_s[b]).start(add=True)

        fire(0, 0)

        def pair(kk, _):
            process(2 * kk, 0)
            process(2 * kk + 1, 1)
            return 0
        lax.fori_loop(0, NCHUNKS // 2, pair, 0)
        drain(NCHUNKS - 1, 1)

        plsc.subcore_barrier()
        pltpu.sync_copy(den_sh.at[pl.ds(r0, ROWS_PER_TILE)],
                        dp_o.at[cid, pl.ds(r0, ROWS_PER_TILE)])
        plsc.subcore_barrier()


def _sc_phase_b_body(ep_qa, ep_aq, ee_qa, ee_aq, den_qa, den_aq,
                     hs_qa, hs_aq, z64,
                     op_qa, op_aq,
                     idx2_0, idx2_1, ee_0, ee_1, den_0, den_1, wbuf,
                     hs_0, hs_1, msg_0, msg_1,
                     out_sh,
                     sem_i0, sem_i1, sem_e0, sem_e1, sem_d0, sem_d1,
                     sem_h0, sem_h1, sem_s0, sem_s1):
    cid = lax.axis_index("c")
    sid = lax.axis_index("s")
    wid = cid * 16 + sid
    r0 = sid * ROWS_PER_TILE
    idx2 = (idx2_0, idx2_1)
    eeb = (ee_0, ee_1)
    denb = (den_0, den_1)
    hsb = (hs_0, hs_1)
    msgb = (msg_0, msg_1)
    sem_i = (sem_i0, sem_i1)
    sem_e = (sem_e0, sem_e1)
    sem_d = (sem_d0, sem_d1)
    sem_h = (sem_h0, sem_h1)
    sem_s = (sem_s0, sem_s1)

    for ep, ee, den, hs, op_o in (
        (ep_qa, ee_qa, den_qa, hs_qa, op_qa),
        (ep_aq, ee_aq, den_aq, hs_aq, op_aq),
    ):
        pltpu.sync_copy(z64.at[pl.ds(r0, ROWS_PER_TILE)],
                        out_sh.at[pl.ds(r0, ROWS_PER_TILE)])
        plsc.subcore_barrier()

        def fire(k, b, ep=ep, ee=ee, den=den, hs=hs):
            g = wid * NCHUNKS_B + k
            base = g * CHUNK_B
            pltpu.sync_copy(ep.at[g], idx2[b])
            pltpu.make_async_copy(hs.at[idx2[b].at[0]], hsb[b], sem_h[b]).start()
            pltpu.make_async_copy(den.at[idx2[b].at[1]], denb[b], sem_d[b]).start()
            pltpu.make_async_copy(ee.at[pl.ds(base, CHUNK_B)], eeb[b],
                                  sem_e[b]).start()

        def wait_scatter(b):
            pltpu.make_async_copy(msgb[b], out_sh.at[idx2[b].at[1]],
                                  sem_s[b]).wait()

        def process(k, b, ee=ee, den=den, hs=hs):
            pltpu.make_async_copy(den.at[idx2[b].at[1]], denb[b], sem_d[b]).wait()
            pltpu.make_async_copy(ee.at[pl.ds(0, CHUNK_B)], eeb[b], sem_e[b]).wait()

            def wrow(i, _):
                wbuf[i, :] = eeb[b][i, :] * 0.125 / (denb[b][i, :] + 1e-16)
                return 0
            lax.fori_loop(0, CHUNK_B, wrow, 0)

            @pl.when(k > 0)
            def _():
                wait_scatter(1 - b)

            @pl.when(k + 1 < NCHUNKS_B)
            def _():
                fire(k + 1, 1 - b)

            pltpu.make_async_copy(hs.at[idx2[b].at[0]], hsb[b], sem_h[b]).wait()

            def edge(e_, _):
                wrow_ = wbuf[e_, :]
                ws = [wrow_[h] for h in range(8)]
                acc = [None] * 4
                for h in range(8):
                    for j in range(4):
                        t = ws[h] * hsb[b][e_, pl.ds(h * 64 + j * 16, 16)]
                        acc[j] = t if h == 0 else acc[j] + t
                for j in range(4):
                    msgb[b][e_, pl.ds(j * 16, 16)] = acc[j]
                return 0
            lax.fori_loop(0, CHUNK_B, edge, 0)
            pltpu.make_async_copy(msgb[b], out_sh.at[idx2[b].at[1]],
                                  sem_s[b]).start(add=True)

        fire(0, 0)

        def pair(kk, _):
            process(2 * kk, 0)
            process(2 * kk + 1, 1)
            return 0
        lax.fori_loop(0, NCHUNKS_B // 2, pair, 0)
        wait_scatter(1)

        plsc.subcore_barrier()
        pltpu.sync_copy(out_sh.at[pl.ds(r0, ROWS_PER_TILE)],
                        op_o.at[cid, pl.ds(r0, ROWS_PER_TILE)])
        plsc.subcore_barrier()


def _tc2_body(opqa, opaq, bqa, baq, wo, bo, out):
    f1 = opaq[0] + opaq[1] + baq[...]
    f2 = opqa[0] + opqa[1] + bqa[...]
    out[...] = (jnp.dot(f1, wo[0:64, :], preferred_element_type=jnp.float32)
                + jnp.dot(f2, wo[64:128, :], preferred_element_type=jnp.float32)
                + bo[...])


def _full(shape):
    return pl.BlockSpec(shape, lambda i: (0,) * len(shape))


@jax.jit
def kernel(x_question, x_answer, edge_index_qa, edge_index_aq,
           W_src_qa, W_dst_qa, att_src_qa, att_dst_qa, bias_qa,
           W_src_aq, W_dst_aq, att_src_aq, att_dst_aq, bias_aq,
           W_out, b_out, ew_qa, ew_aq):
    f32 = jnp.float32
    eye = jnp.eye(HEADS, dtype=f32)

    def amat(att):
        a = (att[:, :, None] * eye[:, None, :]).reshape(HEADS * HID, HEADS)
        return jnp.pad(a, ((0, 0), (0, 16 - HEADS)))

    a_s_qa, a_d_qa = amat(att_src_qa), amat(att_dst_qa)
    a_s_aq, a_d_aq = amat(att_src_aq), amat(att_dst_aq)

    xq = jnp.pad(x_question, ((0, NP - N), (0, 0)))
    xa = jnp.pad(x_answer, ((0, NP - N), (0, 0)))

    pad_idx = jnp.full((EP - E,), PAD_NODE, dtype=jnp.int32)
    es_qa = jnp.concatenate([edge_index_qa[0].astype(jnp.int32), pad_idx])
    ed_qa = jnp.concatenate([edge_index_qa[1].astype(jnp.int32), pad_idx])
    es_aq = jnp.concatenate([edge_index_aq[0].astype(jnp.int32), pad_idx])
    ed_aq = jnp.concatenate([edge_index_aq[1].astype(jnp.int32), pad_idx])
    ep_qa = jnp.stack([es_qa.reshape(-1, CHUNK_B), ed_qa.reshape(-1, CHUNK_B)], 1)
    ep_aq = jnp.stack([es_aq.reshape(-1, CHUNK_B), ed_aq.reshape(-1, CHUNK_B)], 1)
    epa_qa = jnp.stack([es_qa.reshape(-1, CHUNK), ed_qa.reshape(-1, CHUNK)], 1)
    epa_aq = jnp.stack([es_aq.reshape(-1, CHUNK), ed_aq.reshape(-1, CHUNK)], 1)

    tc1 = pl.pallas_call(
        _tc1_body,
        grid=(GRID_N,),
        in_specs=[
            pl.BlockSpec((BN, D_IN), lambda i: (i, 0)),
            pl.BlockSpec((BN, D_IN), lambda i: (i, 0)),
            _full((D_IN, HEADS * HID)), _full((HEADS * HID, 16)),
            _full((D_IN, HEADS * HID)), _full((HEADS * HID, 16)),
            _full((D_IN, HEADS * HID)), _full((HEADS * HID, 16)),
            _full((D_IN, HEADS * HID)), _full((HEADS * HID, 16)),
        ],
        out_specs=[
            pl.BlockSpec((BN, HEADS * HID), lambda i: (i, 0)),
            pl.BlockSpec((BN, 16), lambda i: (i, 0)),
            pl.BlockSpec((BN, 16), lambda i: (i, 0)),
            pl.BlockSpec((BN, HEADS * HID), lambda i: (i, 0)),
            pl.BlockSpec((BN, 16), lambda i: (i, 0)),
            pl.BlockSpec((BN, 16), lambda i: (i, 0)),
        ],
        out_shape=[
            jax.ShapeDtypeStruct((NP, HEADS * HID), f32),
            jax.ShapeDtypeStruct((NP, 16), f32),
            jax.ShapeDtypeStruct((NP, 16), f32),
            jax.ShapeDtypeStruct((NP, HEADS * HID), f32),
            jax.ShapeDtypeStruct((NP, 16), f32),
            jax.ShapeDtypeStruct((NP, 16), f32),
        ],
    )
    hs_qa, al_s_qa, al_d_qa, hs_aq, al_s_aq, al_d_aq = tc1(
        xq, xa, W_src_qa, a_s_qa, W_dst_qa, a_d_qa,
        W_src_aq, a_s_aq, W_dst_aq, a_d_aq)

    mesh = plsc.VectorSubcoreMesh(core_axis_name="c", subcore_axis_name="s")
    z16 = jnp.zeros((NP, 16), f32)
    z64 = jnp.zeros((NP, HID), f32)

    phase_a = pl.kernel(
        _sc_phase_a_body,
        out_type=[
            jax.ShapeDtypeStruct((EP, 16), f32),
            jax.ShapeDtypeStruct((EP, 16), f32),
            jax.ShapeDtypeStruct((2, NP, 16), f32),
            jax.ShapeDtypeStruct((2, NP, 16), f32),
        ],
        mesh=mesh,
        compiler_params=pltpu.CompilerParams(use_tc_tiling_on_sc=False),
        scratch_types=[
            pltpu.VMEM((2, CHUNK), jnp.int32),
            pltpu.VMEM((2, CHUNK), jnp.int32),
            pltpu.VMEM((CHUNK, 16), f32),
            pltpu.VMEM((CHUNK, 16), f32),
            pltpu.VMEM((CHUNK, 16), f32),
            pltpu.VMEM((CHUNK, 16), f32),
            pltpu.VMEM((CHUNK, 16), f32),
            pltpu.VMEM((CHUNK, 16), f32),
            pltpu.VMEM_SHARED((NP, 16), f32),
        ] + [pltpu.SemaphoreType.DMA] * 10,
    )
    ee_qa, ee_aq, dp_qa, dp_aq = phase_a(
        epa_qa, epa_aq, al_s_qa, al_d_qa, al_s_aq, al_d_aq, z16)

    den_qa = dp_qa[0] + dp_qa[1]
    den_aq = dp_aq[0] + dp_aq[1]

    phase_b = pl.kernel(
        _sc_phase_b_body,
        out_type=[
            jax.ShapeDtypeStruct((2, NP, HID), f32),
            jax.ShapeDtypeStruct((2, NP, HID), f32),
        ],
        mesh=mesh,
        compiler_params=pltpu.CompilerParams(use_tc_tiling_on_sc=False),
        scratch_types=[
            pltpu.VMEM((2, CHUNK_B), jnp.int32),
            pltpu.VMEM((2, CHUNK_B), jnp.int32),
            pltpu.VMEM((CHUNK_B, 16), f32),
            pltpu.VMEM((CHUNK_B, 16), f32),
            pltpu.VMEM((CHUNK_B, 16), f32),
            pltpu.VMEM((CHUNK_B, 16), f32),
            pltpu.VMEM((CHUNK_B, 16), f32),
            pltpu.VMEM((CHUNK_B, HEADS * HID), f32),
            pltpu.VMEM((CHUNK_B, HEADS * HID), f32),
            pltpu.VMEM((CHUNK_B, HID), f32),
            pltpu.VMEM((CHUNK_B, HID), f32),
            pltpu.VMEM_SHARED((NP, HID), f32),
        ] + [pltpu.SemaphoreType.DMA] * 10,
    )
    op_qa, op_aq = phase_b(
        ep_qa, ep_aq, ee_qa, ee_aq, den_qa, den_aq,
        hs_qa, hs_aq, z64)

    tc2 = pl.pallas_call(
        _tc2_body,
        grid=(GRID_N,),
        in_specs=[
            pl.BlockSpec((2, BN, HID), lambda i: (0, i, 0)),
            pl.BlockSpec((2, BN, HID), lambda i: (0, i, 0)),
            _full((1, HID)), _full((1, HID)),
            _full((2 * HID, NC_OUT)), _full((1, NC_OUT)),
        ],
        out_specs=pl.BlockSpec((BN, NC_OUT), lambda i: (i, 0)),
        out_shape=jax.ShapeDtypeStruct((NP, NC_OUT), f32),
    )
    preds = tc2(op_qa, op_aq, bias_qa.reshape(1, HID), bias_aq.reshape(1, HID),
                W_out, b_out.reshape(1, NC_OUT))
    return (preds[:N], ew_qa, ew_aq)


# speedup vs baseline: 41.7628x; 1.0316x over previous
import functools
import jax
import jax.numpy as jnp
from jax import lax
from jax.experimental import pallas as pl
from jax.experimental.pallas import tpu as pltpu, tpu_sc as plsc

N = 10000
E = 160000
D_IN = 128
HID = 64
HEADS = 8
NC_OUT = 4

NP = 10240
EP = 163840
PAD_NODE = N

NUM_TILES = 32
EDGES_PER_TILE = EP // NUM_TILES
CHUNK = 128
NCHUNKS = EDGES_PER_TILE // CHUNK
CHUNK_B = 64
NCHUNKS_B = EDGES_PER_TILE // CHUNK_B
ROWS_PER_TILE = NP // 16

BN = 256
GRID_N = NP // BN


def _tc1_body(xq, xa, wsqa, asqa, wdqa, adqa, wsaq, asaq, wdaq, adaq,
              hs_qa, al_s_qa, al_d_qa, hs_aq, al_s_aq, al_d_aq):
    xqb = xq[...]
    xab = xa[...]
    hq = jnp.dot(xqb, wsqa[...], preferred_element_type=jnp.float32)
    hs_qa[...] = hq.astype(jnp.bfloat16)
    al_s_qa[...] = jnp.dot(hq, asqa[...], preferred_element_type=jnp.float32)
    vd_qa = jnp.dot(wdqa[...], adqa[...], preferred_element_type=jnp.float32)
    al_d_qa[...] = jnp.dot(xab, vd_qa, preferred_element_type=jnp.float32)
    ha = jnp.dot(xab, wsaq[...], preferred_element_type=jnp.float32)
    hs_aq[...] = ha.astype(jnp.bfloat16)
    al_s_aq[...] = jnp.dot(ha, asaq[...], preferred_element_type=jnp.float32)
    vd_aq = jnp.dot(wdaq[...], adaq[...], preferred_element_type=jnp.float32)
    al_d_aq[...] = jnp.dot(xqb, vd_aq, preferred_element_type=jnp.float32)


def _sc_phase_a_body(epa_qa, epa_aq, asq, adq, asa, ada, z16,
                     ee_qa, ee_aq, dp_qa, dp_aq,
                     idx2_0, idx2_1, a_0, a_1, b_0, b_1, e_0, e_1, den_sh,
                     sem_i0, sem_i1, sem_a0, sem_a1, sem_b0, sem_b1,
                     sem_w0, sem_w1, sem_s0, sem_s1):
    cid = lax.axis_index("c")
    sid = lax.axis_index("s")
    wid = cid * 16 + sid
    r0 = sid * ROWS_PER_TILE
    idx2 = (idx2_0, idx2_1)
    ab = (a_0, a_1)
    bb = (b_0, b_1)
    eb = (e_0, e_1)
    sem_a = (sem_a0, sem_a1)
    sem_b = (sem_b0, sem_b1)
    sem_w = (sem_w0, sem_w1)
    sem_s = (sem_s0, sem_s1)

    for epa, asrc, adst, ee_o, dp_o in (
        (epa_qa, asq, adq, ee_qa, dp_qa),
        (epa_aq, asa, ada, ee_aq, dp_aq),
    ):
        pltpu.sync_copy(z16.at[pl.ds(r0, ROWS_PER_TILE)],
                        den_sh.at[pl.ds(r0, ROWS_PER_TILE)])
        plsc.subcore_barrier()

        def fire(k, b, epa=epa, asrc=asrc, adst=adst):
            g = wid * NCHUNKS + k
            pltpu.sync_copy(epa.at[g], idx2[b])
            pltpu.make_async_copy(asrc.at[idx2[b].at[0]], ab[b], sem_a[b]).start()
            pltpu.make_async_copy(adst.at[idx2[b].at[1]], bb[b], sem_b[b]).start()

        def drain(k, b, ee_o=ee_o):
            base = (wid * NCHUNKS + k) * CHUNK
            pltpu.make_async_copy(eb[b], ee_o.at[pl.ds(base, CHUNK)],
                                  sem_w[b]).wait()
            pltpu.make_async_copy(eb[b], den_sh.at[idx2[b].at[1]],
                                  sem_s[b]).wait()

        def process(k, b, asrc=asrc, adst=adst, ee_o=ee_o):
            base = (wid * NCHUNKS + k) * CHUNK
            pltpu.make_async_copy(asrc.at[idx2[b].at[0]], ab[b], sem_a[b]).wait()
            pltpu.make_async_copy(adst.at[idx2[b].at[1]], bb[b], sem_b[b]).wait()

            @pl.when(k > 0)
            def _():
                drain(k - 1, 1 - b)

            @pl.when(k + 1 < NCHUNKS)
            def _():
                fire(k + 1, 1 - b)

            def row(i, _):
                s = ab[b][i, :] + bb[b][i, :]
                e = jnp.maximum(s, 0.2 * s)
                eb[b][i, :] = jnp.exp(e)
                return 0
            lax.fori_loop(0, CHUNK, row, 0)
            pltpu.make_async_copy(eb[b], ee_o.at[pl.ds(base, CHUNK)],
                                  sem_w[b]).start()
            pltpu.make_async_copy(eb[b], den_sh.at[idx2[b].at[1]],
                                  sem_s[b]).start(add=True)

        fire(0, 0)

        def pair(kk, _):
            process(2 * kk, 0)
            process(2 * kk + 1, 1)
            return 0
        lax.fori_loop(0, NCHUNKS // 2, pair, 0)
        drain(NCHUNKS - 1, 1)

        plsc.subcore_barrier()
        pltpu.sync_copy(den_sh.at[pl.ds(r0, ROWS_PER_TILE)],
                        dp_o.at[cid, pl.ds(r0, ROWS_PER_TILE)])
        plsc.subcore_barrier()


def _sc_phase_b_body(ep_qa, ep_aq, ee_qa, ee_aq, den_qa, den_aq,
                     hs_qa, hs_aq, z64,
                     op_qa, op_aq,
                     idx2_0, idx2_1, ee_0, ee_1, den_0, den_1, wbuf,
                     hs_0, hs_1, msg_0, msg_1,
                     out_sh,
                     sem_i0, sem_i1, sem_e0, sem_e1, sem_d0, sem_d1,
                     sem_h0, sem_h1, sem_s0, sem_s1):
    cid = lax.axis_index("c")
    sid = lax.axis_index("s")
    wid = cid * 16 + sid
    r0 = sid * ROWS_PER_TILE
    idx2 = (idx2_0, idx2_1)
    eeb = (ee_0, ee_1)
    denb = (den_0, den_1)
    hsb = (hs_0, hs_1)
    msgb = (msg_0, msg_1)
    sem_i = (sem_i0, sem_i1)
    sem_e = (sem_e0, sem_e1)
    sem_d = (sem_d0, sem_d1)
    sem_h = (sem_h0, sem_h1)
    sem_s = (sem_s0, sem_s1)

    for ep, ee, den, hs, op_o in (
        (ep_qa, ee_qa, den_qa, hs_qa, op_qa),
        (ep_aq, ee_aq, den_aq, hs_aq, op_aq),
    ):
        pltpu.sync_copy(z64.at[pl.ds(r0, ROWS_PER_TILE)],
                        out_sh.at[pl.ds(r0, ROWS_PER_TILE)])
        plsc.subcore_barrier()

        def fire(k, b, ep=ep, ee=ee, den=den, hs=hs):
            g = wid * NCHUNKS_B + k
            base = g * CHUNK_B
            pltpu.sync_copy(ep.at[g], idx2[b])
            pltpu.make_async_copy(hs.at[idx2[b].at[0]], hsb[b], sem_h[b]).start()
            pltpu.make_async_copy(den.at[idx2[b].at[1]], denb[b], sem_d[b]).start()
            pltpu.make_async_copy(ee.at[pl.ds(base, CHUNK_B)], eeb[b],
                                  sem_e[b]).start()

        def wait_scatter(b):
            pltpu.make_async_copy(msgb[b], out_sh.at[idx2[b].at[1]],
                                  sem_s[b]).wait()

        def process(k, b, ee=ee, den=den, hs=hs):
            pltpu.make_async_copy(den.at[idx2[b].at[1]], denb[b], sem_d[b]).wait()
            pltpu.make_async_copy(ee.at[pl.ds(0, CHUNK_B)], eeb[b], sem_e[b]).wait()

            def wrow(i, _):
                wbuf[i, :] = eeb[b][i, :] * 0.125 / (denb[b][i, :] + 1e-16)
                return 0
            lax.fori_loop(0, CHUNK_B, wrow, 0)

            @pl.when(k > 0)
            def _():
                wait_scatter(1 - b)

            @pl.when(k + 1 < NCHUNKS_B)
            def _():
                fire(k + 1, 1 - b)

            pltpu.make_async_copy(hs.at[idx2[b].at[0]], hsb[b], sem_h[b]).wait()

            def edge(e_, _):
                wrow_ = wbuf[e_, :]
                ws = [wrow_[h] for h in range(8)]
                acc = [None] * 4
                for h in range(8):
                    for j2 in range(2):
                        raw = hsb[b][e_, pl.ds(h * 64 + j2 * 32, 32)]
                        ti = plsc.bitcast(raw, jnp.int32)
                        fe = plsc.bitcast(ti << 16, jnp.float32)
                        fo = plsc.bitcast(ti & jnp.int32(-65536), jnp.float32)
                        te = ws[h] * fe
                        to = ws[h] * fo
                        qe, qo = 2 * j2, 2 * j2 + 1
                        acc[qe] = te if h == 0 else acc[qe] + te
                        acc[qo] = to if h == 0 else acc[qo] + to
                for q in range(4):
                    msgb[b][e_, pl.ds(q * 16, 16)] = acc[q]
                return 0
            lax.fori_loop(0, CHUNK_B, edge, 0)
            pltpu.make_async_copy(msgb[b], out_sh.at[idx2[b].at[1]],
                                  sem_s[b]).start(add=True)

        fire(0, 0)

        def pair(kk, _):
            process(2 * kk, 0)
            process(2 * kk + 1, 1)
            return 0
        lax.fori_loop(0, NCHUNKS_B // 2, pair, 0)
        wait_scatter(1)

        plsc.subcore_barrier()
        pltpu.sync_copy(out_sh.at[pl.ds(r0, ROWS_PER_TILE)],
                        op_o.at[cid, pl.ds(r0, ROWS_PER_TILE)])
        plsc.subcore_barrier()


def _tc2_body(opqa, opaq, bqa, baq, wo, bo, out):
    f1 = opaq[0] + opaq[1] + baq[...]
    f2 = opqa[0] + opqa[1] + bqa[...]
    out[...] = (jnp.dot(f1, wo[0:64, :], preferred_element_type=jnp.float32)
                + jnp.dot(f2, wo[64:128, :], preferred_element_type=jnp.float32)
                + bo[...])


def _full(shape):
    return pl.BlockSpec(shape, lambda i: (0,) * len(shape))


@jax.jit
def kernel(x_question, x_answer, edge_index_qa, edge_index_aq,
           W_src_qa, W_dst_qa, att_src_qa, att_dst_qa, bias_qa,
           W_src_aq, W_dst_aq, att_src_aq, att_dst_aq, bias_aq,
           W_out, b_out, ew_qa, ew_aq):
    f32 = jnp.float32
    eye = jnp.eye(HEADS, dtype=f32)

    def amat(att):
        a = (att[:, :, None] * eye[:, None, :]).reshape(HEADS * HID, HEADS)
        return jnp.pad(a, ((0, 0), (0, 16 - HEADS)))

    a_s_qa, a_d_qa = amat(att_src_qa), amat(att_dst_qa)
    a_s_aq, a_d_aq = amat(att_src_aq), amat(att_dst_aq)

    l16 = jnp.arange(16)
    perm = jnp.concatenate([2 * l16, 2 * l16 + 1, 32 + 2 * l16, 33 + 2 * l16])
    w_eff = jnp.concatenate([W_out[:64][perm], W_out[64:][perm]], axis=0)
    bias_qa_eff = bias_qa[perm]
    bias_aq_eff = bias_aq[perm]

    xq = jnp.pad(x_question, ((0, NP - N), (0, 0)))
    xa = jnp.pad(x_answer, ((0, NP - N), (0, 0)))

    pad_idx = jnp.full((EP - E,), PAD_NODE, dtype=jnp.int32)
    es_qa = jnp.concatenate([edge_index_qa[0].astype(jnp.int32), pad_idx])
    ed_qa = jnp.concatenate([edge_index_qa[1].astype(jnp.int32), pad_idx])
    es_aq = jnp.concatenate([edge_index_aq[0].astype(jnp.int32), pad_idx])
    ed_aq = jnp.concatenate([edge_index_aq[1].astype(jnp.int32), pad_idx])
    ep_qa = jnp.stack([es_qa.reshape(-1, CHUNK_B), ed_qa.reshape(-1, CHUNK_B)], 1)
    ep_aq = jnp.stack([es_aq.reshape(-1, CHUNK_B), ed_aq.reshape(-1, CHUNK_B)], 1)
    epa_qa = jnp.stack([es_qa.reshape(-1, CHUNK), ed_qa.reshape(-1, CHUNK)], 1)
    epa_aq = jnp.stack([es_aq.reshape(-1, CHUNK), ed_aq.reshape(-1, CHUNK)], 1)

    tc1 = pl.pallas_call(
        _tc1_body,
        grid=(GRID_N,),
        in_specs=[
            pl.BlockSpec((BN, D_IN), lambda i: (i, 0)),
            pl.BlockSpec((BN, D_IN), lambda i: (i, 0)),
            _full((D_IN, HEADS * HID)), _full((HEADS * HID, 16)),
            _full((D_IN, HEADS * HID)), _full((HEADS * HID, 16)),
            _full((D_IN, HEADS * HID)), _full((HEADS * HID, 16)),
            _full((D_IN, HEADS * HID)), _full((HEADS * HID, 16)),
        ],
        out_specs=[
            pl.BlockSpec((BN, HEADS * HID), lambda i: (i, 0)),
            pl.BlockSpec((BN, 16), lambda i: (i, 0)),
            pl.BlockSpec((BN, 16), lambda i: (i, 0)),
            pl.BlockSpec((BN, HEADS * HID), lambda i: (i, 0)),
            pl.BlockSpec((BN, 16), lambda i: (i, 0)),
            pl.BlockSpec((BN, 16), lambda i: (i, 0)),
        ],
        out_shape=[
            jax.ShapeDtypeStruct((NP, HEADS * HID), jnp.bfloat16),
            jax.ShapeDtypeStruct((NP, 16), f32),
            jax.ShapeDtypeStruct((NP, 16), f32),
            jax.ShapeDtypeStruct((NP, HEADS * HID), jnp.bfloat16),
            jax.ShapeDtypeStruct((NP, 16), f32),
            jax.ShapeDtypeStruct((NP, 16), f32),
        ],
    )
    hs_qa, al_s_qa, al_d_qa, hs_aq, al_s_aq, al_d_aq = tc1(
        xq, xa, W_src_qa, a_s_qa, W_dst_qa, a_d_qa,
        W_src_aq, a_s_aq, W_dst_aq, a_d_aq)

    mesh = plsc.VectorSubcoreMesh(core_axis_name="c", subcore_axis_name="s")
    z16 = jnp.zeros((NP, 16), f32)
    z64 = jnp.zeros((NP, HID), f32)

    phase_a = pl.kernel(
        _sc_phase_a_body,
        out_type=[
            jax.ShapeDtypeStruct((EP, 16), f32),
            jax.ShapeDtypeStruct((EP, 16), f32),
            jax.ShapeDtypeStruct((2, NP, 16), f32),
            jax.ShapeDtypeStruct((2, NP, 16), f32),
        ],
        mesh=mesh,
        compiler_params=pltpu.CompilerParams(use_tc_tiling_on_sc=False),
        scratch_types=[
            pltpu.VMEM((2, CHUNK), jnp.int32),
            pltpu.VMEM((2, CHUNK), jnp.int32),
            pltpu.VMEM((CHUNK, 16), f32),
            pltpu.VMEM((CHUNK, 16), f32),
            pltpu.VMEM((CHUNK, 16), f32),
            pltpu.VMEM((CHUNK, 16), f32),
            pltpu.VMEM((CHUNK, 16), f32),
            pltpu.VMEM((CHUNK, 16), f32),
            pltpu.VMEM_SHARED((NP, 16), f32),
        ] + [pltpu.SemaphoreType.DMA] * 10,
    )
    ee_qa, ee_aq, dp_qa, dp_aq = phase_a(
        epa_qa, epa_aq, al_s_qa, al_d_qa, al_s_aq, al_d_aq, z16)

    den_qa = dp_qa[0] + dp_qa[1]
    den_aq = dp_aq[0] + dp_aq[1]

    phase_b = pl.kernel(
        _sc_phase_b_body,
        out_type=[
            jax.ShapeDtypeStruct((2, NP, HID), f32),
            jax.ShapeDtypeStruct((2, NP, HID), f32),
        ],
        mesh=mesh,
        compiler_params=pltpu.CompilerParams(use_tc_tiling_on_sc=False,
                                             needs_layout_passes=False),
        scratch_types=[
            pltpu.VMEM((2, CHUNK_B), jnp.int32),
            pltpu.VMEM((2, CHUNK_B), jnp.int32),
            pltpu.VMEM((CHUNK_B, 16), f32),
            pltpu.VMEM((CHUNK_B, 16), f32),
            pltpu.VMEM((CHUNK_B, 16), f32),
            pltpu.VMEM((CHUNK_B, 16), f32),
            pltpu.VMEM((CHUNK_B, 16), f32),
            pltpu.VMEM((CHUNK_B, HEADS * HID), jnp.bfloat16),
            pltpu.VMEM((CHUNK_B, HEADS * HID), jnp.bfloat16),
            pltpu.VMEM((CHUNK_B, HID), f32),
            pltpu.VMEM((CHUNK_B, HID), f32),
            pltpu.VMEM_SHARED((NP, HID), f32),
        ] + [pltpu.SemaphoreType.DMA] * 10,
    )
    op_qa, op_aq = phase_b(
        ep_qa, ep_aq, ee_qa, ee_aq, den_qa, den_aq,
        hs_qa, hs_aq, z64)

    tc2 = pl.pallas_call(
        _tc2_body,
        grid=(GRID_N,),
        in_specs=[
            pl.BlockSpec((2, BN, HID), lambda i: (0, i, 0)),
            pl.BlockSpec((2, BN, HID), lambda i: (0, i, 0)),
            _full((1, HID)), _full((1, HID)),
            _full((2 * HID, NC_OUT)), _full((1, NC_OUT)),
        ],
        out_specs=pl.BlockSpec((BN, NC_OUT), lambda i: (i, 0)),
        out_shape=jax.ShapeDtypeStruct((NP, NC_OUT), f32),
    )
    preds = tc2(op_qa, op_aq, bias_qa_eff.reshape(1, HID),
                bias_aq_eff.reshape(1, HID), w_eff, b_out.reshape(1, NC_OUT))
    return (preds[:N], ew_qa, ew_aq)
